# baseline (speedup 1.0000x reference)
import functools

import jax
import jax.numpy as jnp
from jax import lax
from jax.experimental import pallas as pl
from jax.experimental.pallas import tpu as pltpu
from jax.experimental.pallas import tpu_sc as plsc

N, C, H, W = 8, 96, 384, 384
P = 524288

NUM_CORES = 2
NUM_SUBCORES = 16
NW = NUM_CORES * NUM_SUBCORES
PPW = P // NW
BC = 128
NCHUNK = PPW // BC
L = 16
CV = C // L


def _sc_body(table, gx_hbm, gy_hbm, idx_hbm, out_hbm,
             gx_v, gy_v, idxp_v,
             r00_v, r01_v, r10_v, r11_v,
             w00_v, w01_v, w10_v, w11_v,
             v00_v, v01_v, v10_v, v11_v,
             out_v, sem):
    cid = lax.axis_index("c")
    sid = lax.axis_index("s")
    wid = sid * NUM_CORES + cid
    base0 = wid * PPW

    def chunk_body(k, carry):
        base = base0 + k * BC
        pltpu.sync_copy(gx_hbm.at[pl.ds(base, BC)], gx_v)
        pltpu.sync_copy(gy_hbm.at[pl.ds(base, BC)], gy_v)
        pltpu.sync_copy(idx_hbm.at[pl.ds(base, BC)], idxp_v)

        for i in range(BC // L):
            sl = pl.ds(i * L, L)
            gx = gx_v[sl]
            gy = gy_v[sl]
            n = idxp_v[sl]
            x = gx * jnp.float32((W - 1) * 0.5) + jnp.float32((W - 1) * 0.5)
            y = gy * jnp.float32((H - 1) * 0.5) + jnp.float32((H - 1) * 0.5)
            x0 = x.astype(jnp.int32)
            y0 = y.astype(jnp.int32)
            x0 = jnp.minimum(jnp.maximum(x0, 0), W - 2)
            y0 = jnp.minimum(jnp.maximum(y0, 0), H - 2)
            fx = x - x0.astype(jnp.float32)
            fy = y - y0.astype(jnp.float32)
            r00 = (n * H + y0) * W + x0
            r00_v[sl] = r00
            r01_v[sl] = r00 + 1
            r10_v[sl] = r00 + W
            r11_v[sl] = r00 + (W + 1)
            ox = jnp.float32(1.0) - fx
            oy = jnp.float32(1.0) - fy
            w00_v[sl] = oy * ox
            w01_v[sl] = oy * fx
            w10_v[sl] = fy * ox
            w11_v[sl] = fy * fx

        d0 = pltpu.async_copy(table.at[r00_v], v00_v, sem)
        d1 = pltpu.async_copy(table.at[r01_v], v01_v, sem)
        d2 = pltpu.async_copy(table.at[r10_v], v10_v, sem)
        d3 = pltpu.async_copy(table.at[r11_v], v11_v, sem)
        d0.wait()
        d1.wait()
        d2.wait()
        d3.wait()

        def pt_body(p, carry2):
            w00 = w00_v[pl.ds(p, L)][0]
            w01 = w01_v[pl.ds(p, L)][0]
            w10 = w10_v[pl.ds(p, L)][0]
            w11 = w11_v[pl.ds(p, L)][0]
            for j in range(CV):
                slj = pl.ds(j * L, L)
                out_v[p, slj] = (w00 * v00_v[p, slj] + w01 * v01_v[p, slj]
                                 + w10 * v10_v[p, slj] + w11 * v11_v[p, slj])
            return carry2

        lax.fori_loop(0, BC, pt_body, 0, unroll=False)
        pltpu.sync_copy(out_v, out_hbm.at[pl.ds(base, BC)])
        return carry

    lax.fori_loop(0, NCHUNK, chunk_body, 0, unroll=False)


def kernel(input, grid, input_indices):
    table = jnp.transpose(input, (0, 2, 3, 1)).reshape(N * H * W, C)
    gx = grid[:, 0]
    gy = grid[:, 1]
    idx = input_indices.astype(jnp.int32)

    mesh = plsc.VectorSubcoreMesh(core_axis_name="c", subcore_axis_name="s")
    f = pl.kernel(
        _sc_body,
        mesh=mesh,
        compiler_params=pltpu.CompilerParams(use_tc_tiling_on_sc=False),
        out_type=jax.ShapeDtypeStruct((P, C), jnp.float32),
        scratch_types=[
            pltpu.VMEM((BC,), jnp.float32),
            pltpu.VMEM((BC,), jnp.float32),
            pltpu.VMEM((BC,), jnp.int32),
            pltpu.VMEM((BC,), jnp.int32),
            pltpu.VMEM((BC,), jnp.int32),
            pltpu.VMEM((BC,), jnp.int32),
            pltpu.VMEM((BC,), jnp.int32),
            pltpu.VMEM((BC + L,), jnp.float32),
            pltpu.VMEM((BC + L,), jnp.float32),
            pltpu.VMEM((BC + L,), jnp.float32),
            pltpu.VMEM((BC + L,), jnp.float32),
            pltpu.VMEM((BC, C), jnp.float32),
            pltpu.VMEM((BC, C), jnp.float32),
            pltpu.VMEM((BC, C), jnp.float32),
            pltpu.VMEM((BC, C), jnp.float32),
            pltpu.VMEM((BC, C), jnp.float32),
            pltpu.SemaphoreType.DMA,
        ],
    )
    return f(table, gx, gy, idx)


# speedup vs baseline: 1.0925x; 1.0925x over previous
import jax
import jax.numpy as jnp
from jax import lax
from jax.experimental import pallas as pl
from jax.experimental.pallas import tpu as pltpu
from jax.experimental.pallas import tpu_sc as plsc

N, C, H, W = 8, 96, 384, 384
P = 524288

NUM_CORES = 2
NUM_SUBCORES = 16
NW = NUM_CORES * NUM_SUBCORES
PPW = P // NW
BC = 128
NCHUNK = PPW // BC
L = 16
CV = C // L


def _sc_body(table, grid_hbm, idx_hbm, out_hbm,
             grid_v, idxp_v,
             r_v,
             w_v,
             v_v,
             out_v,
             sem0, sem1):
    cid = lax.axis_index("c")
    sid = lax.axis_index("s")
    wid = sid * NUM_CORES + cid
    base0 = wid * PPW
    sems = (sem0, sem1)

    def prep_and_fire(k, slot):
        base = base0 + k * BC
        pltpu.sync_copy(grid_hbm.at[pl.ds(2 * base, 2 * BC)], grid_v)
        pltpu.sync_copy(idx_hbm.at[pl.ds(base, BC)], idxp_v)
        for i in range(BC // L):
            sl = pl.ds(i * L, L)
            pair = lax.iota(jnp.int32, L) * 2 + (2 * L * i)
            gx = plsc.load_gather(grid_v, [pair])
            gy = plsc.load_gather(grid_v, [pair + 1])
            n = idxp_v[sl]
            x = gx * jnp.float32((W - 1) * 0.5) + jnp.float32((W - 1) * 0.5)
            y = gy * jnp.float32((H - 1) * 0.5) + jnp.float32((H - 1) * 0.5)
            x0 = x.astype(jnp.int32)
            y0 = y.astype(jnp.int32)
            x0 = jnp.minimum(jnp.maximum(x0, 0), W - 2)
            y0 = jnp.minimum(jnp.maximum(y0, 0), H - 2)
            fx = x - x0.astype(jnp.float32)
            fy = y - y0.astype(jnp.float32)
            r00 = (n * H + y0) * W + x0
            r_v[slot, 0, sl] = r00
            r_v[slot, 1, sl] = r00 + 1
            r_v[slot, 2, sl] = r00 + W
            r_v[slot, 3, sl] = r00 + (W + 1)
            ox = jnp.float32(1.0) - fx
            oy = jnp.float32(1.0) - fy
            w_v[slot, 0, sl] = oy * ox
            w_v[slot, 1, sl] = oy * fx
            w_v[slot, 2, sl] = fy * ox
            w_v[slot, 3, sl] = fy * fx
        for q in range(4):
            pltpu.async_copy(table.at[r_v.at[slot, q]], v_v.at[slot, q],
                             sems[slot])

    def drain_combine_store(k, slot):
        base = base0 + k * BC
        for q in range(4):
            pltpu.make_async_copy(table.at[r_v.at[slot, q]], v_v.at[slot, q],
                                  sems[slot]).wait()

        def pt_body(p, carry):
            pvec = jnp.full((L,), p, dtype=jnp.int32)
            w00 = plsc.load_gather(w_v.at[slot, 0], [pvec])
            w01 = plsc.load_gather(w_v.at[slot, 1], [pvec])
            w10 = plsc.load_gather(w_v.at[slot, 2], [pvec])
            w11 = plsc.load_gather(w_v.at[slot, 3], [pvec])
            for j in range(CV):
                slj = pl.ds(j * L, L)
                out_v[p, slj] = (w00 * v_v[slot, 0, p, slj]
                                 + w01 * v_v[slot, 1, p, slj]
                                 + w10 * v_v[slot, 2, p, slj]
                                 + w11 * v_v[slot, 3, p, slj])
            return carry

        lax.fori_loop(0, BC, pt_body, 0, unroll=False)
        pltpu.sync_copy(out_v, out_hbm.at[pl.ds(base, BC)])

    prep_and_fire(0, 0)

    def pair_body(k2, carry):
        ka = 2 * k2
        prep_and_fire(ka + 1, 1)
        drain_combine_store(ka, 0)

        @pl.when(k2 + 1 < NCHUNK // 2)
        def _():
            prep_and_fire(ka + 2, 0)

        drain_combine_store(ka + 1, 1)
        return carry

    lax.fori_loop(0, NCHUNK // 2, pair_body, 0, unroll=False)


def kernel(input, grid, input_indices):
    table = jnp.transpose(input, (0, 2, 3, 1)).reshape(N * H * W, C)
    grid_flat = grid.reshape(2 * P)
    idx = input_indices.astype(jnp.int32)

    mesh = plsc.VectorSubcoreMesh(core_axis_name="c", subcore_axis_name="s")
    f = pl.kernel(
        _sc_body,
        mesh=mesh,
        compiler_params=pltpu.CompilerParams(
            use_tc_tiling_on_sc=False, needs_layout_passes=False),
        out_type=jax.ShapeDtypeStruct((P, C), jnp.float32),
        scratch_types=[
            pltpu.VMEM((2 * BC,), jnp.float32),
            pltpu.VMEM((BC,), jnp.int32),
            pltpu.VMEM((2, 4, BC), jnp.int32),
            pltpu.VMEM((2, 4, BC), jnp.float32),
            pltpu.VMEM((2, 4, BC, C), jnp.float32),
            pltpu.VMEM((BC, C), jnp.float32),
            pltpu.SemaphoreType.DMA,
            pltpu.SemaphoreType.DMA,
        ],
    )
    return f(table, grid_flat, idx)
